# Initial kernel scaffold; baseline (speedup 1.0000x reference)
import jax
import jax.numpy as jnp
from jax.experimental import pallas as pl


def kernel(x, edge_index, W_mu, W_logvar, W_cls):
    raise NotImplementedError("write your pallas kernel here")


# speedup vs baseline: 2.3195x; 2.3195x over previous
import functools

import jax
import jax.numpy as jnp
from jax import lax
from jax.experimental import pallas as pl
from jax.experimental.pallas import tpu as pltpu
from jax.experimental.pallas import tpu_sc as plsc

N_NODES = 10000
N_EDGES = 320000
NFEAT = 256
NHID = 128
NCLASS = 64

N_TILES = 16
CH = 128
N_PAD = 10240
ROWS_PER_TILE = N_PAD // N_TILES
E_PAD = 327680
CHUNKS = E_PAD // N_TILES // CH


def _make_sc_agg(feat_half, with_deg, mode):
  mesh = plsc.VectorSubcoreMesh(core_axis_name="c", subcore_axis_name="s")
  out_type = [jax.ShapeDtypeStruct((2, N_PAD, feat_half), jnp.float32)]
  if with_deg:
    out_type.append(jax.ShapeDtypeStruct((N_PAD,), jnp.float32))
  scratch = [
      pltpu.VMEM((CH,), jnp.int32),
      pltpu.VMEM((CH,), jnp.int32),
      pltpu.VMEM((CH, feat_half), jnp.float32),
      pltpu.VMEM((CH,), jnp.float32),
      pltpu.VMEM_SHARED((N_PAD, feat_half), jnp.float32),
      pltpu.VMEM_SHARED((N_PAD,), jnp.float32),
      pltpu.SemaphoreType.DMA,
  ]

  def body(xs, srcs, dsts, zrows, *rest):
    if with_deg:
      agg_out, deg_out = rest[0], rest[1]
      idx_s, idx_d, rows, ones, agg_sh, deg_sh, sem = rest[2:]
    else:
      agg_out = rest[0]
      idx_s, idx_d, rows, ones, agg_sh, deg_sh, sem = rest[1:]
    c = lax.axis_index("c")
    s = lax.axis_index("s")
    row0 = s * ROWS_PER_TILE

    pltpu.sync_copy(zrows, agg_sh.at[pl.ds(row0, ROWS_PER_TILE)])
    if with_deg:
      for k in range(ROWS_PER_TILE // feat_half):
        pltpu.sync_copy(zrows.at[k],
                        deg_sh.at[pl.ds(row0 + k * feat_half, feat_half)])
    for k in range(CH // 16):
      ones[pl.ds(k * 16, 16)] = jnp.ones((16,), jnp.float32)
    plsc.subcore_barrier()

    if mode == "feat":
      chunks = CHUNKS
      ebase = s * (E_PAD // N_TILES)
    else:
      chunks = CHUNKS // 2
      ebase = c * (E_PAD // 2) + s * (E_PAD // 2 // N_TILES)

    def step(i, carry):
      off = pl.multiple_of(ebase + i * CH, CH)
      pltpu.sync_copy(srcs.at[pl.ds(off, CH)], idx_s)
      pltpu.sync_copy(dsts.at[pl.ds(off, CH)], idx_d)
      if mode == "feat":
        pltpu.async_copy(xs.at[c].at[idx_s], rows, sem).wait()
      else:
        pltpu.async_copy(xs.at[idx_s], rows, sem).wait()
      pltpu.sync_copy(rows, agg_sh.at[idx_d], add=True)
      if with_deg:
        @pl.when(c == 0)
        def _():
          pltpu.sync_copy(ones, deg_sh.at[idx_d], add=True)
      return carry

    lax.fori_loop(0, chunks, step, 0)
    plsc.subcore_barrier()

    pltpu.sync_copy(agg_sh.at[pl.ds(row0, ROWS_PER_TILE)],
                    agg_out.at[c].at[pl.ds(row0, ROWS_PER_TILE)])
    if with_deg:
      @pl.when(c == 0)
      def _():
        pltpu.sync_copy(deg_sh.at[pl.ds(row0, ROWS_PER_TILE)],
                        deg_out.at[pl.ds(row0, ROWS_PER_TILE)])

  return pl.kernel(body, out_type=out_type, mesh=mesh, scratch_types=scratch)


_R1 = 1000


def _dense1_body(agg_ref, x_ref, deg_ref, w_ref, out_ref):
  agg = agg_ref[...]
  h = jnp.concatenate([agg[0], agg[1]], axis=1) + x_ref[...]
  r = 1.0 / (deg_ref[...] + 1.0)
  h = h * r
  acc = lax.dot_general(h, w_ref[...], (((1,), (1,)), ((), ())),
                        preferred_element_type=jnp.float32)
  out_ref[...] = jnp.maximum(acc, 0.0)


def _dense1(agg, x, degc, wcat):
  return pl.pallas_call(
      _dense1_body,
      grid=(N_NODES // _R1,),
      in_specs=[
          pl.BlockSpec((2, _R1, NHID), lambda i: (0, i, 0)),
          pl.BlockSpec((_R1, NFEAT), lambda i: (i, 0)),
          pl.BlockSpec((_R1, 1), lambda i: (i, 0)),
          pl.BlockSpec((2 * NHID, NFEAT), lambda i: (0, 0)),
      ],
      out_specs=pl.BlockSpec((_R1, 2 * NHID), lambda i: (i, 0)),
      out_shape=jax.ShapeDtypeStruct((N_NODES, 2 * NHID), jnp.float32),
  )(agg, x, degc, wcat)


def _dense2_body(agg_ref, mu_ref, deg_ref, w_ref, out_ref):
  agg = agg_ref[...]
  h = agg[0] + agg[1] + mu_ref[...]
  r = 1.0 / (deg_ref[...] + 1.0)
  h = h * r
  acc = lax.dot_general(h, w_ref[...], (((1,), (1,)), ((), ())),
                        preferred_element_type=jnp.float32)
  out_ref[...] = jnp.maximum(acc, 0.0)


def _dense2(agg2, mu, degc, w_cls):
  return pl.pallas_call(
      _dense2_body,
      grid=(N_NODES // _R1,),
      in_specs=[
          pl.BlockSpec((2, _R1, NHID), lambda i: (0, i, 0)),
          pl.BlockSpec((_R1, NHID), lambda i: (i, 0)),
          pl.BlockSpec((_R1, 1), lambda i: (i, 0)),
          pl.BlockSpec((NCLASS, NHID), lambda i: (0, 0)),
      ],
      out_specs=pl.BlockSpec((_R1, NCLASS), lambda i: (i, 0)),
      out_shape=jax.ShapeDtypeStruct((N_NODES, NCLASS), jnp.float32),
  )(agg2, mu, degc, w_cls)


_RB = 256


def _recons_body(a_ref, b_ref, out_ref):
  z = lax.dot_general(a_ref[...], b_ref[...], (((1,), (1,)), ((), ())),
                      preferred_element_type=jnp.float32)
  out_ref[...] = 1.0 / (1.0 + jnp.exp(-z))


def _recons(mu):
  return pl.pallas_call(
      _recons_body,
      grid=(pl.cdiv(N_NODES, _RB),),
      in_specs=[
          pl.BlockSpec((_RB, NHID), lambda i: (i, 0)),
          pl.BlockSpec((N_NODES, NHID), lambda i: (0, 0)),
      ],
      out_specs=pl.BlockSpec((_RB, N_NODES), lambda i: (i, 0)),
      out_shape=jax.ShapeDtypeStruct((N_NODES, N_NODES), jnp.float32),
  )(mu, mu)


def kernel(x, edge_index, W_mu, W_logvar, W_cls):
  src = edge_index[0].astype(jnp.int32)
  dst = edge_index[1].astype(jnp.int32)
  npad_e = E_PAD - N_EDGES
  pad_idx = jnp.full((npad_e,), N_PAD - 1, jnp.int32)
  src_p = jnp.concatenate([src, pad_idx])
  dst_p = jnp.concatenate([dst, pad_idx])

  x_pad = jnp.zeros((N_PAD, NFEAT), jnp.float32).at[:N_NODES].set(x)
  xs = jnp.stack([x_pad[:, :NHID], x_pad[:, NHID:]])
  zrows1 = jnp.zeros((ROWS_PER_TILE, NHID), jnp.float32)

  sc1 = _make_sc_agg(NHID, with_deg=True, mode="feat")
  agg, deg = sc1(xs, src_p, dst_p, zrows1)
  degc = deg[:N_NODES, None]

  wcat = jnp.concatenate([W_mu, W_logvar], axis=0)
  ml = _dense1(agg, x, degc, wcat)
  mu = ml[:, :NHID]
  logvar = ml[:, NHID:]

  mu_pad = jnp.zeros((N_PAD, NHID), jnp.float32).at[:N_NODES].set(mu)

  sc2 = _make_sc_agg(NHID, with_deg=False, mode="edge")
  agg2 = sc2(mu_pad, src_p, dst_p, zrows1)
  if isinstance(agg2, (tuple, list)):
    agg2 = agg2[0]

  rst = _dense2(agg2, mu, degc, W_cls)
  recons = _recons(mu)
  return (rst, recons, mu, logvar)


# speedup vs baseline: 3.5561x; 1.5331x over previous
import functools

import jax
import jax.numpy as jnp
from jax import lax
from jax.experimental import pallas as pl
from jax.experimental.pallas import tpu as pltpu
from jax.experimental.pallas import tpu_sc as plsc

N_NODES = 10000
N_EDGES = 320000
NFEAT = 256
NHID = 128
NCLASS = 64

N_TILES = 16
CH = 128
N_PAD = 10240
ROWS_PER_TILE = N_PAD // N_TILES
E_PAD = 327680
CHUNKS = E_PAD // N_TILES // CH


G = 16


def _make_sc_agg(feat_half, with_deg, tc_tiling=True):
  mesh = plsc.VectorSubcoreMesh(core_axis_name="c", subcore_axis_name="s")
  chunks = CHUNKS
  ngroups = chunks // G
  assert ngroups % 2 == 0
  out_type = [jax.ShapeDtypeStruct((2, N_PAD, feat_half), jnp.float32)]
  if with_deg:
    out_type.append(jax.ShapeDtypeStruct((N_PAD,), jnp.float32))
  scratch = [
      pltpu.VMEM((G, CH), jnp.int32),
      pltpu.VMEM((G, CH), jnp.int32),
      pltpu.VMEM((G, CH), jnp.int32),
      pltpu.VMEM((G, CH), jnp.int32),
      pltpu.VMEM((CH, feat_half), jnp.float32),
      pltpu.VMEM((CH, feat_half), jnp.float32),
      pltpu.VMEM((CH,), jnp.float32),
      pltpu.VMEM_SHARED((N_PAD, feat_half), jnp.float32),
      pltpu.SemaphoreType.DMA,
      pltpu.SemaphoreType.DMA,
      pltpu.SemaphoreType.DMA,
      pltpu.SemaphoreType.DMA,
  ]
  if with_deg:
    scratch.append(pltpu.VMEM_SHARED((N_PAD,), jnp.float32))

  def body(xs, srcs, dsts, zrows, *rest):
    if with_deg:
      agg_out, deg_out = rest[0], rest[1]
      scr = rest[2:]
    else:
      agg_out = rest[0]
      scr = rest[1:]
    deg_sh = None
    if with_deg:
      (sa, da, sb, db, rows_a, rows_b, ones, agg_sh,
       sem_a, sem_b, isem_a, isem_b, deg_sh) = scr
    else:
      (sa, da, sb, db, rows_a, rows_b, ones, agg_sh,
       sem_a, sem_b, isem_a, isem_b) = scr
    c = lax.axis_index("c")
    s = lax.axis_index("s")
    row0 = s * ROWS_PER_TILE

    idx = ((sa, da, isem_a), (sb, db, isem_b))
    rows = ((rows_a, sem_a), (rows_b, sem_b))
    cbase = s * ngroups

    def start_idx(g, b):
      off = pl.multiple_of((cbase + g) * G, 8)
      sbuf, dbuf, isem = idx[b]
      pltpu.async_copy(srcs.at[pl.ds(off, G)], sbuf, isem)
      pltpu.async_copy(dsts.at[pl.ds(off, G)], dbuf, isem)

    def wait_idx(b):
      sbuf, dbuf, isem = idx[b]
      pltpu.make_async_copy(srcs.at[pl.ds(0, G)], sbuf, isem).wait()
      pltpu.make_async_copy(dsts.at[pl.ds(0, G)], dbuf, isem).wait()

    def gather(idx_row, buf, sem):
      pltpu.async_copy(xs.at[c].at[idx_row], buf, sem)

    def gwait(idx_row, buf, sem):
      pltpu.make_async_copy(xs.at[c].at[idx_row], buf, sem).wait()

    def consume(idx_row, buf):
      pltpu.sync_copy(buf, agg_sh.at[idx_row], add=True)
      if with_deg:
        @pl.when(c == 0)
        def _():
          pltpu.sync_copy(ones, deg_sh.at[idx_row], add=True)

    pltpu.sync_copy(zrows, agg_sh.at[pl.ds(row0, ROWS_PER_TILE)])
    if with_deg:
      for k in range(ROWS_PER_TILE // feat_half):
        pltpu.sync_copy(zrows.at[k],
                        deg_sh.at[pl.ds(row0 + k * feat_half, feat_half)])
    for k in range(CH // 16):
      ones[pl.ds(k * 16, 16)] = jnp.ones((16,), jnp.float32)

    start_idx(0, 0)
    start_idx(1, 1)
    plsc.subcore_barrier()
    wait_idx(0)
    gather(sa.at[0], rows_a, sem_a)

    def do_group(b):
      sbuf, dbuf, _ = idx[b]
      nsbuf = idx[1 - b][0]
      for k in range(G):
        buf, sem = rows[k % 2]
        nbuf, nsem = rows[(k + 1) % 2]
        gwait(sbuf.at[k], buf, sem)
        if k == G - 1:
          wait_idx(1 - b)
          gather(nsbuf.at[0], nbuf, nsem)
        else:
          gather(sbuf.at[k + 1], nbuf, nsem)
        consume(dbuf.at[k], buf)

    def pair(p, carry):
      do_group(0)
      start_idx(jnp.minimum(2 * p + 2, ngroups - 1), 0)
      do_group(1)
      start_idx(jnp.minimum(2 * p + 3, ngroups - 1), 1)
      return carry

    lax.fori_loop(0, ngroups // 2, pair, 0)
    gwait(sa.at[0], rows_a, sem_a)
    wait_idx(1)
    plsc.subcore_barrier()

    pltpu.sync_copy(agg_sh.at[pl.ds(row0, ROWS_PER_TILE)],
                    agg_out.at[c].at[pl.ds(row0, ROWS_PER_TILE)])
    if with_deg:
      @pl.when(c == 0)
      def _():
        pltpu.sync_copy(deg_sh.at[pl.ds(row0, ROWS_PER_TILE)],
                        deg_out.at[pl.ds(row0, ROWS_PER_TILE)])

  params = None if tc_tiling else pltpu.CompilerParams(use_tc_tiling_on_sc=False)
  return pl.kernel(body, out_type=out_type, mesh=mesh, scratch_types=scratch,
                   compiler_params=params)


_R1 = 1000


def _dense1_body(agg_ref, x_ref, deg_ref, w_ref, wcls_ref, out_ref, y_ref):
  agg = agg_ref[...]
  h = jnp.concatenate([agg[0], agg[1]], axis=1) + x_ref[...]
  r = 1.0 / (deg_ref[...] + 1.0)
  h = h * r
  acc = lax.dot_general(h, w_ref[...], (((1,), (1,)), ((), ())),
                        preferred_element_type=jnp.float32)
  m = jnp.maximum(acc, 0.0)
  out_ref[...] = m
  y_ref[...] = lax.dot_general(m[:, :NHID], wcls_ref[...],
                               (((1,), (1,)), ((), ())),
                               preferred_element_type=jnp.float32)


def _dense1(agg, x, degc, wcat, w_cls):
  return pl.pallas_call(
      _dense1_body,
      grid=(N_NODES // _R1,),
      in_specs=[
          pl.BlockSpec((2, _R1, NHID), lambda i: (0, i, 0)),
          pl.BlockSpec((_R1, NFEAT), lambda i: (i, 0)),
          pl.BlockSpec((_R1, 1), lambda i: (i, 0)),
          pl.BlockSpec((2 * NHID, NFEAT), lambda i: (0, 0)),
          pl.BlockSpec((NCLASS, NHID), lambda i: (0, 0)),
      ],
      out_specs=[
          pl.BlockSpec((_R1, 2 * NHID), lambda i: (i, 0)),
          pl.BlockSpec((_R1, NCLASS), lambda i: (i, 0)),
      ],
      out_shape=[
          jax.ShapeDtypeStruct((N_NODES, 2 * NHID), jnp.float32),
          jax.ShapeDtypeStruct((N_NODES, NCLASS), jnp.float32),
      ],
  )(agg, x, degc, wcat, w_cls)


def _dense2_body(agg_ref, y_ref, deg_ref, out_ref):
  agg = agg_ref[...]
  h = jnp.concatenate([agg[0], agg[1]], axis=1) + y_ref[...]
  r = 1.0 / (deg_ref[...] + 1.0)
  out_ref[...] = jnp.maximum(h * r, 0.0)


def _dense2(agg2, y, degc):
  return pl.pallas_call(
      _dense2_body,
      grid=(N_NODES // _R1,),
      in_specs=[
          pl.BlockSpec((2, _R1, NCLASS // 2), lambda i: (0, i, 0)),
          pl.BlockSpec((_R1, NCLASS), lambda i: (i, 0)),
          pl.BlockSpec((_R1, 1), lambda i: (i, 0)),
      ],
      out_specs=pl.BlockSpec((_R1, NCLASS), lambda i: (i, 0)),
      out_shape=jax.ShapeDtypeStruct((N_NODES, NCLASS), jnp.float32),
  )(agg2, y, degc)


_RB = 256


def _recons_body(a_ref, b_ref, out_ref):
  z = lax.dot_general(a_ref[...], b_ref[...], (((1,), (1,)), ((), ())),
                      preferred_element_type=jnp.float32)
  out_ref[...] = 1.0 / (1.0 + jnp.exp(-z))


def _recons(mu):
  return pl.pallas_call(
      _recons_body,
      grid=(pl.cdiv(N_NODES, _RB),),
      in_specs=[
          pl.BlockSpec((_RB, NHID), lambda i: (i, 0)),
          pl.BlockSpec((N_NODES, NHID), lambda i: (0, 0)),
      ],
      out_specs=pl.BlockSpec((_RB, N_NODES), lambda i: (i, 0)),
      out_shape=jax.ShapeDtypeStruct((N_NODES, N_NODES), jnp.float32),
  )(mu, mu)


def kernel(x, edge_index, W_mu, W_logvar, W_cls):
  src = edge_index[0].astype(jnp.int32)
  dst = edge_index[1].astype(jnp.int32)
  npad_e = E_PAD - N_EDGES
  pad_idx = jnp.full((npad_e,), N_PAD - 1, jnp.int32)
  src_p = jnp.concatenate([src, pad_idx]).reshape(E_PAD // CH, CH)
  dst_p = jnp.concatenate([dst, pad_idx]).reshape(E_PAD // CH, CH)

  x_pad = jnp.zeros((N_PAD, NFEAT), jnp.float32).at[:N_NODES].set(x)
  xs = jnp.stack([x_pad[:, :NHID], x_pad[:, NHID:]])
  zrows1 = jnp.zeros((ROWS_PER_TILE, NHID), jnp.float32)

  sc1 = _make_sc_agg(NHID, with_deg=True)
  agg, deg = sc1(xs, src_p, dst_p, zrows1)
  degc = deg[:N_NODES, None]

  wcat = jnp.concatenate([W_mu, W_logvar], axis=0)
  ml, y = _dense1(agg, x, degc, wcat, W_cls)
  mu = ml[:, :NHID]
  logvar = ml[:, NHID:]

  y_pad = jnp.zeros((N_PAD, NCLASS), jnp.float32).at[:N_NODES].set(y)
  ys = jnp.stack([y_pad[:, :NCLASS // 2], y_pad[:, NCLASS // 2:]])
  zrows2 = jnp.zeros((ROWS_PER_TILE, NCLASS // 2), jnp.float32)

  sc2 = _make_sc_agg(NCLASS // 2, with_deg=False, tc_tiling=False)
  agg2 = sc2(ys, src_p, dst_p, zrows2)
  if isinstance(agg2, (tuple, list)):
    agg2 = agg2[0]

  rst = _dense2(agg2, y, degc)
  recons = _recons(mu)
  return (rst, recons, mu, logvar)
